# baseline (speedup 1.0000x reference)
import functools

import jax
import jax.numpy as jnp
from jax import lax
from jax.experimental import pallas as pl
from jax.experimental.pallas import tpu as pltpu
from jax.experimental.pallas import tpu_sc as plsc

BATCH = 16384
D = 32
NC = 2
NS = 16
NW = NC * NS
BPW = BATCH // NW
CHUNK = 128
NCHUNK = BPW // CHUNK
GROUPS = CHUNK // 16


def _body(uid_ref, iid_ref, utab, itab, out_hbm,
          uidx_v, iidx_v, urows, irows, out_v, sem):
    wid = lax.axis_index("s") * NC + lax.axis_index("c")
    row0 = wid * NCHUNK

    pltpu.sync_copy(uid_ref.at[pl.ds(row0, NCHUNK)], uidx_v)
    pltpu.sync_copy(iid_ref.at[pl.ds(row0, NCHUNK)], iidx_v)

    copies = []
    for c in range(NCHUNK):
        copies.append(pltpu.async_copy(
            utab.at[uidx_v.at[c]], urows.at[pl.ds(c * CHUNK, CHUNK)], sem))
        copies.append(pltpu.async_copy(
            itab.at[iidx_v.at[c]], irows.at[pl.ds(c * CHUNK, CHUNK)], sem))
    for cp in copies:
        cp.wait()

    lanes = lax.iota(jnp.int32, 16)

    def g_body(g, _):
        rows = g * 16 + lanes
        acc = jnp.zeros((16,), jnp.float32)
        for j in range(D):
            col = jnp.full((16,), j, jnp.int32)
            uv = plsc.load_gather(urows, [rows, col])
            iv = plsc.load_gather(irows, [rows, col])
            acc = acc + uv * iv
        plsc.store_scatter(out_v, [rows], acc)
        return 0

    lax.fori_loop(0, BPW // 16, g_body, 0)

    pltpu.sync_copy(out_v, out_hbm.at[pl.ds(wid * BPW, BPW)])


def kernel(user_ids, item_ids, user_table, item_table):
    uid2d = user_ids.astype(jnp.int32).reshape(BATCH // CHUNK, CHUNK)
    iid2d = item_ids.astype(jnp.int32).reshape(BATCH // CHUNK, CHUNK)
    mesh = plsc.VectorSubcoreMesh(core_axis_name="c", subcore_axis_name="s")
    run = pl.kernel(
        _body,
        mesh=mesh,
        out_type=jax.ShapeDtypeStruct((BATCH,), jnp.float32),
        scratch_types=[
            pltpu.VMEM((NCHUNK, CHUNK), jnp.int32),
            pltpu.VMEM((NCHUNK, CHUNK), jnp.int32),
            pltpu.VMEM((BPW, D), jnp.float32),
            pltpu.VMEM((BPW, D), jnp.float32),
            pltpu.VMEM((BPW,), jnp.float32),
            pltpu.SemaphoreType.DMA,
        ],
        compiler_params=pltpu.CompilerParams(
            needs_layout_passes=False, use_tc_tiling_on_sc=False),
    )
    return run(uid2d, iid2d, user_table, item_table)


# speedup vs baseline: 1.0070x; 1.0070x over previous
import jax
import jax.numpy as jnp
from jax import lax
from jax.experimental import pallas as pl
from jax.experimental.pallas import tpu as pltpu
from jax.experimental.pallas import tpu_sc as plsc

BATCH = 16384
D = 32
PACK = 4
NC = 2
NS = 16
NW = NC * NS
BPW = BATCH // NW
CHUNK = 128
NCHUNK = BPW // CHUNK
GROUPS = CHUNK // 16


def _body(uid_ref, iid_ref, utab, itab, out_hbm,
          uidx_v, iidx_v, u4_v, i4_v, uo_v, io_v,
          ubufs, ibufs, out_v, sem):
    wid = lax.axis_index("s") * NC + lax.axis_index("c")
    base = wid * BPW

    pltpu.sync_copy(uid_ref.at[pl.ds(base, BPW)], uidx_v)
    pltpu.sync_copy(iid_ref.at[pl.ds(base, BPW)], iidx_v)

    for s in range(BPW // 16):
        sl = pl.ds(s * 16, 16)
        u = uidx_v[sl]
        u4_v[sl] = u >> 2
        uo_v[sl] = (u & 3) * D
        i = iidx_v[sl]
        i4_v[sl] = i >> 2
        io_v[sl] = (i & 3) * D

    def fire(c):
        csl = pl.ds(c * CHUNK, CHUNK)
        ub, ib = ubufs[c % 2], ibufs[c % 2]
        return (pltpu.async_copy(utab.at[u4_v.at[csl]], ub, sem),
                pltpu.async_copy(itab.at[i4_v.at[csl]], ib, sem))

    lanes = lax.iota(jnp.int32, 16)
    inflight = {0: fire(0), 1: fire(1)}

    for c in range(NCHUNK):
        for cp in inflight.pop(c):
            cp.wait()
        ub, ib = ubufs[c % 2], ibufs[c % 2]

        def g_body(g, _, ub=ub, ib=ib, c=c):
            rows_l = g * 16 + lanes
            rows_g = c * CHUNK + rows_l
            uo = plsc.load_gather(uo_v, [rows_g])
            io = plsc.load_gather(io_v, [rows_g])
            acc = jnp.zeros((16,), jnp.float32)
            for t in range(D):
                colp = (t + lanes) & (D - 1)
                uval = plsc.load_gather(ub, [rows_l, uo + colp])
                ival = plsc.load_gather(ib, [rows_l, io + colp])
                acc = acc + uval * ival
            plsc.store_scatter(out_v, [rows_g], acc)
            return 0

        lax.fori_loop(0, GROUPS, g_body, 0)
        if c + 2 < NCHUNK:
            inflight[c + 2] = fire(c + 2)

    pltpu.sync_copy(out_v, out_hbm.at[pl.ds(base, BPW)])


def kernel(user_ids, item_ids, user_table, item_table):
    ut4 = user_table.reshape(-1, PACK * D)
    it4 = item_table.reshape(-1, PACK * D)
    mesh = plsc.VectorSubcoreMesh(core_axis_name="c", subcore_axis_name="s")
    run = pl.kernel(
        _body,
        mesh=mesh,
        out_type=jax.ShapeDtypeStruct((BATCH,), jnp.float32),
        scratch_types=dict(
            uidx_v=pltpu.VMEM((BPW,), jnp.int32),
            iidx_v=pltpu.VMEM((BPW,), jnp.int32),
            u4_v=pltpu.VMEM((BPW,), jnp.int32),
            i4_v=pltpu.VMEM((BPW,), jnp.int32),
            uo_v=pltpu.VMEM((BPW,), jnp.int32),
            io_v=pltpu.VMEM((BPW,), jnp.int32),
            ubufs=[pltpu.VMEM((CHUNK, PACK * D), jnp.float32)] * 2,
            ibufs=[pltpu.VMEM((CHUNK, PACK * D), jnp.float32)] * 2,
            out_v=pltpu.VMEM((BPW,), jnp.float32),
            sem=pltpu.SemaphoreType.DMA,
        ),
        compiler_params=pltpu.CompilerParams(
            needs_layout_passes=False, use_tc_tiling_on_sc=True),
    )
    return run(user_ids.astype(jnp.int32), item_ids.astype(jnp.int32),
               ut4, it4)


# speedup vs baseline: 3.9170x; 3.8896x over previous
import jax
import jax.numpy as jnp
from jax import lax
from jax.experimental import pallas as pl
from jax.experimental.pallas import tpu as pltpu
from jax.experimental.pallas import tpu_sc as plsc

BATCH = 16384
D = 32
LANE = 128
NC = 2
NS = 16
NW = NC * NS
BPW = BATCH // NW
NBUF = 4


def _body(uid_ref, iid_ref, utab, itab, out_hbm,
          uidx_v, iidx_v, ubufs, ibufs, out_v, usems, isems):
    wid = lax.axis_index("s") * NC + lax.axis_index("c")
    base = wid * BPW

    pltpu.sync_copy(uid_ref.at[pl.ds(base, BPW)], uidx_v.at[pl.ds(0, BPW)])
    pltpu.sync_copy(iid_ref.at[pl.ds(base, BPW)], iidx_v.at[pl.ds(0, BPW)])

    lanes = lax.iota(jnp.int32, 16)

    def ids_at(e):
        uvec = plsc.load_gather(uidx_v, [jnp.full((16,), 0, jnp.int32) + e])
        ivec = plsc.load_gather(iidx_v, [jnp.full((16,), 0, jnp.int32) + e])
        return uvec[0], ivec[0]

    def fire(e, s):
        uid, iid = ids_at(e)
        uoff = pl.multiple_of((uid >> 7) << 7, LANE)
        ioff = pl.multiple_of((iid >> 7) << 7, LANE)
        pltpu.async_copy(utab.at[:, pl.ds(uoff, LANE)], ubufs[s], usems[s])
        pltpu.async_copy(itab.at[:, pl.ds(ioff, LANE)], ibufs[s], isems[s])

    def drain(s):
        pltpu.make_async_copy(utab.at[:, pl.ds(0, LANE)], ubufs[s],
                              usems[s]).wait()
        pltpu.make_async_copy(itab.at[:, pl.ds(0, LANE)], ibufs[s],
                              isems[s]).wait()

    def compute(e, s):
        uid, iid = ids_at(e)
        lu = jnp.full((16,), 0, jnp.int32) + (uid & (LANE - 1))
        li = jnp.full((16,), 0, jnp.int32) + (iid & (LANE - 1))
        u_lo = plsc.load_gather(ubufs[s], [lanes, lu])
        u_hi = plsc.load_gather(ubufs[s], [lanes + 16, lu])
        i_lo = plsc.load_gather(ibufs[s], [lanes, li])
        i_hi = plsc.load_gather(ibufs[s], [lanes + 16, li])
        p = u_lo * i_lo + u_hi * i_hi
        sv = jnp.sum(p)
        plsc.store_scatter(out_v, [jnp.full((16,), 0, jnp.int32) + e],
                           jnp.full((16,), 0.0, jnp.float32) + sv,
                           mask=lanes == 0)

    for s in range(NBUF):
        fire(s, s)

    def i_body(i, _):
        for s in range(NBUF):
            e = i * NBUF + s
            drain(s)
            compute(e, s)
            fire(e + NBUF, s)
        return 0

    lax.fori_loop(0, BPW // NBUF - 1, i_body, 0)
    for s in range(NBUF):
        e = BPW - NBUF + s
        drain(s)
        compute(e, s)

    pltpu.sync_copy(out_v, out_hbm.at[pl.ds(base, BPW)])


def kernel(user_ids, item_ids, user_table, item_table):
    utT = user_table.T
    itT = item_table.T
    mesh = plsc.VectorSubcoreMesh(core_axis_name="c", subcore_axis_name="s")
    run = pl.kernel(
        _body,
        mesh=mesh,
        out_type=jax.ShapeDtypeStruct((BATCH,), jnp.float32),
        scratch_types=dict(
            uidx_v=pltpu.VMEM((BPW + 16,), jnp.int32),
            iidx_v=pltpu.VMEM((BPW + 16,), jnp.int32),
            ubufs=[pltpu.VMEM((D, LANE), jnp.float32)] * NBUF,
            ibufs=[pltpu.VMEM((D, LANE), jnp.float32)] * NBUF,
            out_v=pltpu.VMEM((BPW,), jnp.float32),
            usems=[pltpu.SemaphoreType.DMA] * NBUF,
            isems=[pltpu.SemaphoreType.DMA] * NBUF,
        ),
        compiler_params=pltpu.CompilerParams(
            needs_layout_passes=False, use_tc_tiling_on_sc=True),
    )
    return run(user_ids.astype(jnp.int32), item_ids.astype(jnp.int32),
               utT, itT)


# speedup vs baseline: 4.0207x; 1.0265x over previous
import jax
import jax.numpy as jnp
from jax import lax
from jax.experimental import pallas as pl
from jax.experimental.pallas import tpu as pltpu
from jax.experimental.pallas import tpu_sc as plsc

BATCH = 16384
D = 32
LANE = 128
NC = 2
NS = 16
NW = NC * NS
BPW = BATCH // NW
NBUF = 8


def _body(uid_ref, iid_ref, utab, itab, out_hbm,
          uidx_v, iidx_v, ubufs, ibufs, out_v, usems, isems):
    wid = lax.axis_index("s") * NC + lax.axis_index("c")
    base = wid * BPW

    pltpu.sync_copy(uid_ref.at[pl.ds(base, BPW)], uidx_v.at[pl.ds(0, BPW)])
    pltpu.sync_copy(iid_ref.at[pl.ds(base, BPW)], iidx_v.at[pl.ds(0, BPW)])

    lanes = lax.iota(jnp.int32, 16)

    def ids_at(e):
        uvec = plsc.load_gather(uidx_v, [jnp.full((16,), 0, jnp.int32) + e])
        ivec = plsc.load_gather(iidx_v, [jnp.full((16,), 0, jnp.int32) + e])
        return uvec[0], ivec[0]

    def fire(e, s):
        uid, iid = ids_at(e)
        uoff = pl.multiple_of((uid >> 7) << 7, LANE)
        ioff = pl.multiple_of((iid >> 7) << 7, LANE)
        pltpu.async_copy(utab.at[:, pl.ds(uoff, LANE)], ubufs[s], usems[s])
        pltpu.async_copy(itab.at[:, pl.ds(ioff, LANE)], ibufs[s], isems[s])

    def drain(s):
        pltpu.make_async_copy(utab.at[:, pl.ds(0, LANE)], ubufs[s],
                              usems[s]).wait()
        pltpu.make_async_copy(itab.at[:, pl.ds(0, LANE)], ibufs[s],
                              isems[s]).wait()

    def compute(e, s):
        uid, iid = ids_at(e)
        lu = jnp.full((16,), 0, jnp.int32) + (uid & (LANE - 1))
        li = jnp.full((16,), 0, jnp.int32) + (iid & (LANE - 1))
        u_lo = plsc.load_gather(ubufs[s], [lanes, lu])
        u_hi = plsc.load_gather(ubufs[s], [lanes + 16, lu])
        i_lo = plsc.load_gather(ibufs[s], [lanes, li])
        i_hi = plsc.load_gather(ibufs[s], [lanes + 16, li])
        p = u_lo * i_lo + u_hi * i_hi
        sv = jnp.sum(p)
        plsc.store_scatter(out_v, [jnp.full((16,), 0, jnp.int32) + e],
                           jnp.full((16,), 0.0, jnp.float32) + sv,
                           mask=lanes == 0)

    for s in range(NBUF):
        fire(s, s)

    def i_body(i, _):
        for s in range(NBUF):
            e = i * NBUF + s
            drain(s)
            compute(e, s)
            fire(e + NBUF, s)
        return 0

    lax.fori_loop(0, BPW // NBUF - 1, i_body, 0)
    for s in range(NBUF):
        e = BPW - NBUF + s
        drain(s)
        compute(e, s)

    pltpu.sync_copy(out_v, out_hbm.at[pl.ds(base, BPW)])


def kernel(user_ids, item_ids, user_table, item_table):
    utT = user_table.T
    itT = item_table.T
    mesh = plsc.VectorSubcoreMesh(core_axis_name="c", subcore_axis_name="s")
    run = pl.kernel(
        _body,
        mesh=mesh,
        out_type=jax.ShapeDtypeStruct((BATCH,), jnp.float32),
        scratch_types=dict(
            uidx_v=pltpu.VMEM((BPW + 16,), jnp.int32),
            iidx_v=pltpu.VMEM((BPW + 16,), jnp.int32),
            ubufs=[pltpu.VMEM((D, LANE), jnp.float32)] * NBUF,
            ibufs=[pltpu.VMEM((D, LANE), jnp.float32)] * NBUF,
            out_v=pltpu.VMEM((BPW,), jnp.float32),
            usems=[pltpu.SemaphoreType.DMA] * NBUF,
            isems=[pltpu.SemaphoreType.DMA] * NBUF,
        ),
        compiler_params=pltpu.CompilerParams(
            needs_layout_passes=False, use_tc_tiling_on_sc=True),
    )
    return run(user_ids.astype(jnp.int32), item_ids.astype(jnp.int32),
               utT, itT)
